# baseline (speedup 1.0000x reference)
import jax
import jax.numpy as jnp
from jax.experimental import pallas as pl
from jax.experimental.pallas import tpu as pltpu


def _packed_body(x2_ref, w_ref, b_ref, o_ref):
    x2 = jnp.maximum(x2_ref[...], 0.0)
    w = w_ref[...]
    b = b_ref[...]
    dn = (((1,), (0,)), ((), ()))
    y_lo = jax.lax.dot_general(x2[:, :128], w, dn,
                               preferred_element_type=jnp.float32) + b
    y_hi = jax.lax.dot_general(x2[:, 128:], w, dn,
                               preferred_element_type=jnp.float32) + b
    o_ref[...] = jnp.concatenate([y_lo, y_hi], axis=-1)


def kernel(x_subject, x_region, edge_index_sr, edge_index_rr, edge_attr_sr,
           edge_attr_rr, sage_Wl0, sage_bl0, sage_Wr0, gcn_W0, gcn_b0,
           sage_Wl1, sage_bl1, sage_Wr1, gcn_W1, gcn_b1, lin_W, lin_b):
    m, d = x_subject.shape
    out_dim = lin_W.shape[1]
    x2 = x_subject.reshape(m // 2, 2 * d)
    o2 = pl.pallas_call(
        _packed_body,
        out_shape=jax.ShapeDtypeStruct((m // 2, 2 * out_dim), jnp.float32),
    )(x2, lin_W, lin_b.reshape(1, out_dim))
    return o2.reshape(m, out_dim)


# speedup vs baseline: 1.3312x; 1.3312x over previous
import jax
import jax.numpy as jnp
from jax.experimental import pallas as pl
from jax.experimental.pallas import tpu as pltpu

_CH = 10
_HALF = 500


def _body(x_hbm, w_ref, b_ref, o_ref, xbuf, sems):
    x2 = x_hbm.reshape(5000, 256)
    copies = []
    for i in range(_CH):
        c = pltpu.make_async_copy(
            x2.at[pl.ds(i * _HALF, _HALF), :], xbuf.at[i], sems.at[i])
        c.start()
        copies.append(c)
    w = w_ref[...]
    b = b_ref[...]
    dn = (((1,), (0,)), ((), ()))
    for i in range(_CH):
        copies[i].wait()
        xc = xbuf[i]
        ye = jax.lax.dot_general(jnp.maximum(xc[:, :128], 0.0), w, dn,
                                 preferred_element_type=jnp.float32) + b
        yo = jax.lax.dot_general(jnp.maximum(xc[:, 128:], 0.0), w, dn,
                                 preferred_element_type=jnp.float32) + b
        o_ref[pl.ds(i * _HALF, _HALF), 0:64] = ye
        o_ref[pl.ds(i * _HALF, _HALF), 64:128] = yo


def kernel(x_subject, x_region, edge_index_sr, edge_index_rr, edge_attr_sr,
           edge_attr_rr, sage_Wl0, sage_bl0, sage_Wr0, gcn_W0, gcn_b0,
           sage_Wl1, sage_bl1, sage_Wr1, gcn_W1, gcn_b1, lin_W, lin_b):
    m, d = x_subject.shape
    out_dim = lin_W.shape[1]
    o2 = pl.pallas_call(
        _body,
        in_specs=[
            pl.BlockSpec(memory_space=pltpu.MemorySpace.HBM),
            pl.BlockSpec(memory_space=pltpu.MemorySpace.VMEM),
            pl.BlockSpec(memory_space=pltpu.MemorySpace.VMEM),
        ],
        out_specs=pl.BlockSpec(memory_space=pltpu.MemorySpace.VMEM),
        out_shape=jax.ShapeDtypeStruct((m // 2, 2 * out_dim), jnp.float32),
        scratch_shapes=[
            pltpu.VMEM((_CH, _HALF, 2 * d), jnp.float32),
            pltpu.SemaphoreType.DMA((_CH,)),
        ],
    )(x_subject, lin_W, lin_b.reshape(1, out_dim))
    return o2.reshape(m, out_dim)


# speedup vs baseline: 1.4913x; 1.1203x over previous
import jax
import jax.numpy as jnp
from jax.experimental import pallas as pl
from jax.experimental.pallas import tpu as pltpu

_CH = 4
_ROWS = 2500


def _body(x_hbm, w64_ref, b_ref, o_hbm, xbuf, obuf, isems, osems):
    cin = []
    for i in range(_CH):
        c = pltpu.make_async_copy(
            x_hbm.at[pl.ds(i * _ROWS, _ROWS), :], xbuf.at[i], isems.at[i])
        c.start()
        cin.append(c)
    w64 = w64_ref[...]
    r = jax.lax.broadcasted_iota(jnp.int32, (128, 64), 0)
    c2 = jax.lax.broadcasted_iota(jnp.int32, (128, 64), 1)
    a_even = jnp.where(r == 2 * c2, 1.0, 0.0).astype(jnp.float32)
    a_odd = jnp.where(r == 2 * c2 + 1, 1.0, 0.0).astype(jnp.float32)
    dn = (((1,), (0,)), ((), ()))
    w = (jax.lax.dot_general(a_even, w64[:, :64], dn,
                             preferred_element_type=jnp.float32)
         + jax.lax.dot_general(a_odd, w64[:, 64:], dn,
                               preferred_element_type=jnp.float32))
    b = b_ref[...]
    cout = []
    for i in range(_CH):
        cin[i].wait()
        obuf[i] = jax.lax.dot_general(
            jnp.maximum(xbuf[i], 0.0), w, dn,
            preferred_element_type=jnp.float32) + b
        co = pltpu.make_async_copy(
            obuf.at[i], o_hbm.at[pl.ds(i * _ROWS, _ROWS), :], osems.at[i])
        co.start()
        cout.append(co)
    for co in cout:
        co.wait()


def kernel(x_subject, x_region, edge_index_sr, edge_index_rr, edge_attr_sr,
           edge_attr_rr, sage_Wl0, sage_bl0, sage_Wr0, gcn_W0, gcn_b0,
           sage_Wl1, sage_bl1, sage_Wr1, gcn_W1, gcn_b1, lin_W, lin_b):
    m, d = x_subject.shape
    out_dim = lin_W.shape[1]
    return pl.pallas_call(
        _body,
        in_specs=[
            pl.BlockSpec(memory_space=pltpu.MemorySpace.HBM),
            pl.BlockSpec(memory_space=pltpu.MemorySpace.VMEM),
            pl.BlockSpec(memory_space=pltpu.MemorySpace.VMEM),
        ],
        out_specs=pl.BlockSpec(memory_space=pltpu.MemorySpace.HBM),
        out_shape=jax.ShapeDtypeStruct((m, out_dim), jnp.float32),
        scratch_shapes=[
            pltpu.VMEM((_CH, _ROWS, d), jnp.float32),
            pltpu.VMEM((_CH, _ROWS, out_dim), jnp.float32),
            pltpu.SemaphoreType.DMA((_CH,)),
            pltpu.SemaphoreType.DMA((_CH,)),
        ],
    )(x_subject, lin_W.reshape(out_dim, d), lin_b.reshape(1, out_dim))


# speedup vs baseline: 1.6262x; 1.0905x over previous
import jax
import jax.numpy as jnp
from jax.experimental import pallas as pl


def _relu_matmul_bias_kernel(x_ref, w_ref, b_ref, o_ref):
    x = jnp.maximum(x_ref[...], 0.0)
    acc = jax.lax.dot_general(
        x, w_ref[...], (((1,), (0,)), ((), ())),
        preferred_element_type=jnp.float32,
    )
    o_ref[...] = acc + b_ref[...]


def kernel(x_subject, x_region, edge_index_sr, edge_index_rr, edge_attr_sr,
           edge_attr_rr, sage_Wl0, sage_bl0, sage_Wr0, gcn_W0, gcn_b0,
           sage_Wl1, sage_bl1, sage_Wr1, gcn_W1, gcn_b1, lin_W, lin_b):
    m, d = x_subject.shape
    out_dim = lin_W.shape[1]
    return pl.pallas_call(
        _relu_matmul_bias_kernel,
        out_shape=jax.ShapeDtypeStruct((m, out_dim), jnp.float32),
    )(x_subject, lin_W, lin_b.reshape(1, out_dim))
